# Initial kernel scaffold; baseline (speedup 1.0000x reference)
import jax
import jax.numpy as jnp
from jax.experimental import pallas as pl


def kernel(actions, enc_W1, enc_b1, enc_W2, enc_b2, enc_W3, enc_b3, dec_W1, dec_b1, dec_W2, dec_b2, dec_W3, dec_b3, embed):
    raise NotImplementedError("write your pallas kernel here")


# speedup vs baseline: 1.3135x; 1.3135x over previous
import jax
import jax.numpy as jnp
from jax.experimental import pallas as pl
from jax.experimental.pallas import tpu as pltpu

B = 2048
H = 16
A = 32
IN = 512
NL = 1024
NE = 1024
NG = 8
CW = 0.25
CHUNK = 512


def _dot(a, b, trans_b=False):
    dims = (((1,), (1 if trans_b else 0,)), ((), ()))
    return jax.lax.dot_general(a, b, dims, preferred_element_type=jnp.float32)


def _enc_body(x_ref, w1_ref, b1_ref, w2_ref, b2_ref, w3_ref, b3_ref, z_ref):
    z = jnp.maximum(_dot(x_ref[...], w1_ref[...]) + b1_ref[...], 0.0)
    z = jnp.maximum(_dot(z, w2_ref[...]) + b2_ref[...], 0.0)
    z_ref[...] = _dot(z, w3_ref[...]) + b3_ref[...]


def _vq_body(z_ref, emb_ref, idx_ref, quant_ref, loss_ref, res_scr, quant_scr):
    c = pl.program_id(0)
    g = pl.program_id(1)

    @pl.when(jnp.logical_and(c == 0, g == 0))
    def _():
        loss_ref[...] = jnp.zeros_like(loss_ref)

    @pl.when(g == 0)
    def _():
        res_scr[...] = z_ref[...]
        quant_scr[...] = jnp.zeros_like(quant_scr)

    r = res_scr[...]
    E = emb_ref[0]
    s = _dot(r, E, trans_b=True)
    rn = jnp.sum(r * r, axis=1, keepdims=True)
    en = jnp.sum(E * E, axis=1)[None, :]
    dist = (rn - 2.0 * s) + en
    m = jnp.min(dist, axis=1, keepdims=True)
    iota = jax.lax.broadcasted_iota(jnp.int32, dist.shape, 1)
    idx = jnp.min(jnp.where(dist == m, iota, NE), axis=1)
    oh = (iota == idx[:, None]).astype(jnp.float32)
    q = jax.lax.dot_general(oh, E, (((1,), (0,)), ((), ())),
                            precision=jax.lax.Precision.HIGHEST,
                            preferred_element_type=jnp.float32)
    q_st = r + (q - r)
    quant = quant_scr[...] + q_st
    quant_scr[...] = quant
    res_scr[...] = z_ref[...] - quant
    d = r - q
    loss_ref[...] += (jnp.sum(d * d) * ((1.0 + CW) / (B * NL))).reshape(1, 1)
    idx_ref[...] = idx.reshape(1, 1, CHUNK)

    @pl.when(g == NG - 1)
    def _():
        quant_ref[...] = quant


def _dec_body(q_ref, x_ref, w1_ref, b1_ref, w2_ref, b2_ref, w3_ref, b3_ref,
              rec_ref, loss_ref):
    c = pl.program_id(0)

    @pl.when(c == 0)
    def _():
        loss_ref[...] = jnp.zeros_like(loss_ref)

    h = jnp.maximum(_dot(q_ref[...], w1_ref[...]) + b1_ref[...], 0.0)
    h = jnp.maximum(_dot(h, w2_ref[...]) + b2_ref[...], 0.0)
    rec = _dot(h, w3_ref[...]) + b3_ref[...]
    rec_ref[...] = rec
    d = rec - x_ref[...]
    loss_ref[...] += (jnp.sum(d * d) * (1.0 / (B * IN))).reshape(1, 1)


def _full(shape):
    return pl.BlockSpec(shape, lambda *_: tuple(0 for _ in shape))


def kernel(actions, enc_W1, enc_b1, enc_W2, enc_b2, enc_W3, enc_b3,
           dec_W1, dec_b1, dec_W2, dec_b2, dec_W3, dec_b3, embed):
    x = actions.reshape(B, IN)
    nc = B // CHUNK

    z = pl.pallas_call(
        _enc_body,
        grid=(nc,),
        in_specs=[
            pl.BlockSpec((CHUNK, IN), lambda c: (c, 0)),
            _full((IN, NL)), _full((1, NL)),
            _full((NL, NL)), _full((1, NL)),
            _full((NL, NL)), _full((1, NL)),
        ],
        out_specs=pl.BlockSpec((CHUNK, NL), lambda c: (c, 0)),
        out_shape=jax.ShapeDtypeStruct((B, NL), jnp.float32),
    )(x, enc_W1, enc_b1.reshape(1, NL), enc_W2, enc_b2.reshape(1, NL),
      enc_W3, enc_b3.reshape(1, NL))

    idx, quant, vq_loss = pl.pallas_call(
        _vq_body,
        grid=(nc, NG),
        in_specs=[
            pl.BlockSpec((CHUNK, NL), lambda c, g: (c, 0)),
            pl.BlockSpec((1, NE, NL), lambda c, g: (g, 0, 0)),
        ],
        out_specs=[
            pl.BlockSpec((1, 1, CHUNK), lambda c, g: (g, 0, c)),
            pl.BlockSpec((CHUNK, NL), lambda c, g: (c, 0)),
            pl.BlockSpec((1, 1), lambda c, g: (0, 0)),
        ],
        out_shape=[
            jax.ShapeDtypeStruct((NG, 1, B), jnp.int32),
            jax.ShapeDtypeStruct((B, NL), jnp.float32),
            jax.ShapeDtypeStruct((1, 1), jnp.float32),
        ],
        scratch_shapes=[
            pltpu.VMEM((CHUNK, NL), jnp.float32),
            pltpu.VMEM((CHUNK, NL), jnp.float32),
        ],
    )(z, embed)

    rec, rec_loss = pl.pallas_call(
        _dec_body,
        grid=(nc,),
        in_specs=[
            pl.BlockSpec((CHUNK, NL), lambda c: (c, 0)),
            pl.BlockSpec((CHUNK, IN), lambda c: (c, 0)),
            _full((NL, NL)), _full((1, NL)),
            _full((NL, NL)), _full((1, NL)),
            _full((NL, IN)), _full((1, IN)),
        ],
        out_specs=[
            pl.BlockSpec((CHUNK, IN), lambda c: (c, 0)),
            pl.BlockSpec((1, 1), lambda c: (0, 0)),
        ],
        out_shape=[
            jax.ShapeDtypeStruct((B, IN), jnp.float32),
            jax.ShapeDtypeStruct((1, 1), jnp.float32),
        ],
    )(quant, x, dec_W1, dec_b1.reshape(1, NL), dec_W2, dec_b2.reshape(1, NL),
      dec_W3, dec_b3.reshape(1, IN))

    reconstructed = rec.reshape(B, H, A)
    indices = idx.reshape(NG, B).T
    total_loss = vq_loss[0, 0] + rec_loss[0, 0]
    return (reconstructed, indices, total_loss)


# speedup vs baseline: 1.8683x; 1.4223x over previous
import jax
import jax.numpy as jnp
from jax.experimental import pallas as pl
from jax.experimental.pallas import tpu as pltpu

B = 2048
H = 16
A = 32
IN = 512
NL = 1024
NE = 1024
NG = 8
CW = 0.25
CHUNK = 512


def _dot(a, b, trans_b=False):
    dims = (((1,), (1 if trans_b else 0,)), ((), ()))
    return jax.lax.dot_general(a, b, dims, preferred_element_type=jnp.float32)


def _enc_body(x_ref, w1_ref, b1_ref, w2_ref, b2_ref, w3_ref, b3_ref, z_ref):
    z = jnp.maximum(_dot(x_ref[...], w1_ref[...]) + b1_ref[...], 0.0)
    z = jnp.maximum(_dot(z, w2_ref[...]) + b2_ref[...], 0.0)
    z_ref[...] = _dot(z, w3_ref[...]) + b3_ref[...]


def _vq_body(z_ref, emb_ref, idx_ref, quant_ref, loss_ref, res_scr, quant_scr):
    c = pl.program_id(0)
    g = pl.program_id(1)

    @pl.when(jnp.logical_and(c == 0, g == 0))
    def _():
        loss_ref[...] = jnp.zeros_like(loss_ref)

    @pl.when(g == 0)
    def _():
        res_scr[...] = z_ref[...]
        quant_scr[...] = jnp.zeros_like(quant_scr)

    r = res_scr[...]
    E = emb_ref[0]
    s = _dot(r, E, trans_b=True)
    rn = jnp.sum(r * r, axis=1, keepdims=True)
    en = jnp.sum(E * E, axis=1)[None, :]
    dist = (rn - 2.0 * s) + en
    m = jnp.min(dist, axis=1, keepdims=True)
    iota = jax.lax.broadcasted_iota(jnp.int32, dist.shape, 1)
    idx = jnp.min(jnp.where(dist == m, iota, NE), axis=1)
    e1 = E.astype(jnp.bfloat16)
    r1 = E - e1.astype(jnp.float32)
    e2 = r1.astype(jnp.bfloat16)
    e3 = (r1 - e2.astype(jnp.float32)).astype(jnp.bfloat16)
    oh = (iota == idx[:, None]).astype(jnp.bfloat16)
    q = (_dot(oh, e1) + _dot(oh, e2)) + _dot(oh, e3)
    q_st = r + (q - r)
    quant = quant_scr[...] + q_st
    quant_scr[...] = quant
    res_scr[...] = z_ref[...] - quant
    d = r - q
    loss_ref[...] += (jnp.sum(d * d) * ((1.0 + CW) / (B * NL))).reshape(1, 1)
    idx_ref[...] = idx.reshape(1, 1, CHUNK)

    @pl.when(g == NG - 1)
    def _():
        quant_ref[...] = quant


def _dec_body(q_ref, x_ref, w1_ref, b1_ref, w2_ref, b2_ref, w3_ref, b3_ref,
              rec_ref, loss_ref):
    c = pl.program_id(0)

    @pl.when(c == 0)
    def _():
        loss_ref[...] = jnp.zeros_like(loss_ref)

    h = jnp.maximum(_dot(q_ref[...], w1_ref[...]) + b1_ref[...], 0.0)
    h = jnp.maximum(_dot(h, w2_ref[...]) + b2_ref[...], 0.0)
    rec = _dot(h, w3_ref[...]) + b3_ref[...]
    rec_ref[...] = rec
    d = rec - x_ref[...]
    loss_ref[...] += (jnp.sum(d * d) * (1.0 / (B * IN))).reshape(1, 1)


def _full(shape):
    return pl.BlockSpec(shape, lambda *_: tuple(0 for _ in shape))


def kernel(actions, enc_W1, enc_b1, enc_W2, enc_b2, enc_W3, enc_b3,
           dec_W1, dec_b1, dec_W2, dec_b2, dec_W3, dec_b3, embed):
    x = actions.reshape(B, IN)
    nc = B // CHUNK

    z = pl.pallas_call(
        _enc_body,
        grid=(nc,),
        in_specs=[
            pl.BlockSpec((CHUNK, IN), lambda c: (c, 0)),
            _full((IN, NL)), _full((1, NL)),
            _full((NL, NL)), _full((1, NL)),
            _full((NL, NL)), _full((1, NL)),
        ],
        out_specs=pl.BlockSpec((CHUNK, NL), lambda c: (c, 0)),
        out_shape=jax.ShapeDtypeStruct((B, NL), jnp.float32),
    )(x, enc_W1, enc_b1.reshape(1, NL), enc_W2, enc_b2.reshape(1, NL),
      enc_W3, enc_b3.reshape(1, NL))

    idx, quant, vq_loss = pl.pallas_call(
        _vq_body,
        grid=(nc, NG),
        in_specs=[
            pl.BlockSpec((CHUNK, NL), lambda c, g: (c, 0)),
            pl.BlockSpec((1, NE, NL), lambda c, g: (g, 0, 0)),
        ],
        out_specs=[
            pl.BlockSpec((1, 1, CHUNK), lambda c, g: (g, 0, c)),
            pl.BlockSpec((CHUNK, NL), lambda c, g: (c, 0)),
            pl.BlockSpec((1, 1), lambda c, g: (0, 0)),
        ],
        out_shape=[
            jax.ShapeDtypeStruct((NG, 1, B), jnp.int32),
            jax.ShapeDtypeStruct((B, NL), jnp.float32),
            jax.ShapeDtypeStruct((1, 1), jnp.float32),
        ],
        scratch_shapes=[
            pltpu.VMEM((CHUNK, NL), jnp.float32),
            pltpu.VMEM((CHUNK, NL), jnp.float32),
        ],
    )(z, embed)

    rec, rec_loss = pl.pallas_call(
        _dec_body,
        grid=(nc,),
        in_specs=[
            pl.BlockSpec((CHUNK, NL), lambda c: (c, 0)),
            pl.BlockSpec((CHUNK, IN), lambda c: (c, 0)),
            _full((NL, NL)), _full((1, NL)),
            _full((NL, NL)), _full((1, NL)),
            _full((NL, IN)), _full((1, IN)),
        ],
        out_specs=[
            pl.BlockSpec((CHUNK, IN), lambda c: (c, 0)),
            pl.BlockSpec((1, 1), lambda c: (0, 0)),
        ],
        out_shape=[
            jax.ShapeDtypeStruct((B, IN), jnp.float32),
            jax.ShapeDtypeStruct((1, 1), jnp.float32),
        ],
    )(quant, x, dec_W1, dec_b1.reshape(1, NL), dec_W2, dec_b2.reshape(1, NL),
      dec_W3, dec_b3.reshape(1, IN))

    reconstructed = rec.reshape(B, H, A)
    indices = idx.reshape(NG, B).T
    total_loss = vq_loss[0, 0] + rec_loss[0, 0]
    return (reconstructed, indices, total_loss)
